# Initial kernel scaffold; baseline (speedup 1.0000x reference)
import jax
import jax.numpy as jnp
from jax.experimental import pallas as pl


def kernel(x, edge_index, W1_l, b1_l, W1_r, W2_l, b2_l, W2_r):
    raise NotImplementedError("write your pallas kernel here")


# speedup vs baseline: 5.0063x; 5.0063x over previous
import functools

import jax
import jax.numpy as jnp
from jax import lax
from jax.experimental import pallas as pl
from jax.experimental.pallas import tpu as pltpu
from jax.experimental.pallas import tpu_sc as plsc

NC = 2
NS = 16
NW = NC * NS
LANES = 16
CHUNK = 128


def _make_sc_agg(n_nodes, n_pad, d_pad, chunks_per_worker):
    rows_per_tile = n_pad // NS
    mesh = plsc.VectorSubcoreMesh(core_axis_name="c", subcore_axis_name="s",
                                  num_cores=NC, num_subcores=NS)

    @functools.partial(
        pl.kernel,
        mesh=mesh,
        out_type=jax.ShapeDtypeStruct((NC, n_pad, d_pad), jnp.float32),
        scratch_types=[
            pltpu.VMEM((chunks_per_worker, CHUNK), jnp.int32),
            pltpu.VMEM((chunks_per_worker, CHUNK), jnp.int32),
            pltpu.VMEM((CHUNK, d_pad), jnp.float32),
            pltpu.VMEM_SHARED((n_pad, d_pad), jnp.float32),
            pltpu.SemaphoreType.DMA,
        ],
        compiler_params=pltpu.CompilerParams(use_tc_tiling_on_sc=False),
    )
    def sc_agg(feat_hbm, src_hbm, dst_hbm, out_hbm, src_v, dst_v, rows_v,
               acc, sem):
        cid = lax.axis_index("c")
        sid = lax.axis_index("s")
        wid = sid * NC + cid

        pltpu.sync_copy(src_hbm.at[wid], src_v)
        pltpu.sync_copy(dst_hbm.at[wid], dst_v)

        def zero_row(j, carry):
            for k in range(d_pad // LANES):
                rows_v[j, pl.ds(k * LANES, LANES)] = jnp.zeros(
                    (LANES,), jnp.float32)
            return carry
        lax.fori_loop(0, CHUNK, zero_row, 0)

        base = pl.multiple_of(sid * rows_per_tile, 8)
        off = 0
        while off < rows_per_tile:
            sz = min(CHUNK, rows_per_tile - off)
            pltpu.sync_copy(rows_v.at[pl.ds(0, sz)],
                            acc.at[pl.ds(base + off, sz)])
            off += sz
        plsc.subcore_barrier()

        def chunk_body(c, carry):
            pltpu.async_copy(feat_hbm.at[src_v.at[c]], rows_v, sem).wait()
            pltpu.sync_copy(rows_v, acc.at[dst_v.at[c]], add=True)
            return carry
        lax.fori_loop(0, chunks_per_worker, chunk_body, 0)
        plsc.subcore_barrier()

        pltpu.sync_copy(acc.at[pl.ds(base, rows_per_tile)],
                        out_hbm.at[cid, pl.ds(base, rows_per_tile)])

    return sc_agg


def _tc_sage_body(relu, pad_ones, bm, d, d_pad,
                  p_ref, x_ref, wl_ref, wr_ref, b_ref, o_ref):
    agg = p_ref[0] + p_ref[1]
    cnt = jnp.maximum(agg[:, d:d + 1], 1.0)
    mean = agg[:, :d] / cnt
    h = (jnp.dot(mean, wl_ref[...], preferred_element_type=jnp.float32)
         + jnp.dot(x_ref[:, :d], wr_ref[...],
                   preferred_element_type=jnp.float32)
         + b_ref[...])
    if relu:
        h = jnp.maximum(h, 0.0)
    if pad_ones:
        o_ref[:, :d] = h
        o_ref[:, d:] = jnp.ones((bm, d_pad - d), jnp.float32)
    else:
        o_ref[...] = h


def _tc_sage(p, x_feat, wl_t, wr_t, b, relu, pad_ones, bm):
    n, d = x_feat.shape[0], wl_t.shape[0]
    d_pad = p.shape[2]
    d_out = d_pad if pad_ones else d
    grid = n // bm
    return pl.pallas_call(
        functools.partial(_tc_sage_body, relu, pad_ones, bm, d, d_pad),
        grid=(grid,),
        in_specs=[
            pl.BlockSpec((NC, bm, d_pad), lambda i: (0, i, 0)),
            pl.BlockSpec((bm, x_feat.shape[1]), lambda i: (i, 0)),
            pl.BlockSpec((d, d), lambda i: (0, 0)),
            pl.BlockSpec((d, d), lambda i: (0, 0)),
            pl.BlockSpec((1, d), lambda i: (0, 0)),
        ],
        out_specs=pl.BlockSpec((bm, d_out), lambda i: (i, 0)),
        out_shape=jax.ShapeDtypeStruct((n, d_out), jnp.float32),
    )(p, x_feat, wl_t, wr_t, b)


def kernel(x, edge_index, W1_l, b1_l, W1_r, W2_l, b2_l, W2_r):
    n, d = x.shape
    e = edge_index.shape[1]
    assert e % NW == 0
    ep = e // NW
    cpw = -(-ep // CHUNK)
    pad_e = cpw * CHUNK - ep
    d_pad = d + LANES
    rows_per_tile = -(-(n + 1) // (NS * 8)) * 8
    n_pad = rows_per_tile * NS

    src = edge_index[0].astype(jnp.int32).reshape(NW, ep)
    dst = edge_index[1].astype(jnp.int32).reshape(NW, ep)
    if pad_e:
        src = jnp.concatenate(
            [src, jnp.zeros((NW, pad_e), jnp.int32)], axis=1)
        dst = jnp.concatenate(
            [dst, jnp.full((NW, pad_e), n, jnp.int32)], axis=1)
    src = src.reshape(NW, cpw, CHUNK)
    dst = dst.reshape(NW, cpw, CHUNK)

    xp = jnp.concatenate([x, jnp.ones((n, d_pad - d), jnp.float32)], axis=1)

    sc_agg = _make_sc_agg(n, n_pad, d_pad, cpw)
    p1 = sc_agg(xp, src, dst)
    hp = _tc_sage(p1, xp, W1_l.T, W1_r.T, b1_l.reshape(1, d),
                  relu=True, pad_ones=True, bm=1000)
    p2 = sc_agg(hp, src, dst)
    out = _tc_sage(p2, hp, W2_l.T, W2_r.T, b2_l.reshape(1, d),
                   relu=False, pad_ones=False, bm=1000)
    return out
